# initial kernel scaffold (unmeasured)
import jax
import jax.numpy as jnp
from jax import lax
from jax.experimental import pallas as pl
from jax.experimental.pallas import tpu as pltpu

N_DEV = 8
M = 1024
N = 1024
CHUNK = M // N_DEV


def kernel(x, w_mat):
    def body(
        x_ref,
        w_ref,
        out_ref,
        pacc_ref,
        send_ref,
        rs_recv_ref,
        ag_recv_ref,
        rs_send_sems,
        rs_recv_sems,
        ag_send_sems,
        ag_recv_sems,
    ):
        my = lax.axis_index("i")
        right = lax.rem(my + 1, N_DEV)

        pacc_ref[...] = jnp.dot(
            x_ref[...], w_ref[...], preferred_element_type=jnp.float32
        )

        for s in range(N_DEV - 1):
            c = lax.rem(my - s + N_DEV, N_DEV)
            chunk = pacc_ref[pl.ds(c * CHUNK, CHUNK), :]
            if s == 0:
                send_ref[...] = chunk
            else:
                send_ref[...] = chunk + rs_recv_ref[s - 1]
            rdma = pltpu.make_async_remote_copy(
                src_ref=send_ref,
                dst_ref=rs_recv_ref.at[s],
                send_sem=rs_send_sems.at[s],
                recv_sem=rs_recv_sems.at[s],
                device_id=(right,),
                device_id_type=pl.DeviceIdType.MESH,
            )
            rdma.start()
            rdma.wait()

        c_own = lax.rem(my + 1, N_DEV)
        acc = pacc_ref[pl.ds(c_own * CHUNK, CHUNK), :] + rs_recv_ref[N_DEV - 2]
        y = acc * jax.nn.sigmoid(acc)
        out_ref[pl.ds(c_own * CHUNK, CHUNK), :] = y
        send_ref[...] = y

        for s in range(N_DEV - 1):
            if s > 0:
                send_ref[...] = ag_recv_ref[s - 1]
            rdma = pltpu.make_async_remote_copy(
                src_ref=send_ref,
                dst_ref=ag_recv_ref.at[s],
                send_sem=ag_send_sems.at[s],
                recv_sem=ag_recv_sems.at[s],
                device_id=(right,),
                device_id_type=pl.DeviceIdType.MESH,
            )
            rdma.start()
            rdma.wait()
            c_in = lax.rem(my - s + N_DEV, N_DEV)
            out_ref[pl.ds(c_in * CHUNK, CHUNK), :] = ag_recv_ref[s]

    return pl.pallas_call(
        body,
        out_shape=jax.ShapeDtypeStruct((M, N), jnp.float32),
        in_specs=[
            pl.BlockSpec(memory_space=pltpu.VMEM),
            pl.BlockSpec(memory_space=pltpu.VMEM),
        ],
        out_specs=pl.BlockSpec(memory_space=pltpu.VMEM),
        scratch_shapes=[
            pltpu.VMEM((M, N), jnp.float32),
            pltpu.VMEM((CHUNK, N), jnp.float32),
            pltpu.VMEM((N_DEV - 1, CHUNK, N), jnp.float32),
            pltpu.VMEM((N_DEV - 1, CHUNK, N), jnp.float32),
            pltpu.SemaphoreType.DMA((N_DEV - 1,)),
            pltpu.SemaphoreType.DMA((N_DEV - 1,)),
            pltpu.SemaphoreType.DMA((N_DEV - 1,)),
            pltpu.SemaphoreType.DMA((N_DEV - 1,)),
        ],
        compiler_params=pltpu.CompilerParams(collective_id=0),
    )(x, w_mat)


# baseline (device time: 116973 ns/iter reference)
import jax
import jax.numpy as jnp
from jax import lax
from jax.experimental import pallas as pl
from jax.experimental.pallas import tpu as pltpu

N_DEV = 8
M = 1024
N = 1024
CHUNK = M // N_DEV


def kernel(x, w_mat):
    def body(
        x_ref,
        w_ref,
        out_ref,
        pacc_ref,
        send_ref,
        rs_recv_ref,
        ag_recv_ref,
        rs_send_sems,
        rs_recv_sems,
        ag_send_sems,
        ag_recv_sems,
    ):
        my = lax.axis_index("i")
        right = lax.rem(my + 1, N_DEV)

        pacc_ref[...] = jnp.dot(
            x_ref[...], w_ref[...], preferred_element_type=jnp.float32
        )

        for s in range(N_DEV - 1):
            c = lax.rem(my - s + N_DEV, N_DEV)
            chunk = pacc_ref[pl.ds(c * CHUNK, CHUNK), :]
            if s == 0:
                send_ref[...] = chunk
            else:
                send_ref[...] = chunk + rs_recv_ref[s - 1]
            rdma = pltpu.make_async_remote_copy(
                src_ref=send_ref,
                dst_ref=rs_recv_ref.at[s],
                send_sem=rs_send_sems.at[s],
                recv_sem=rs_recv_sems.at[s],
                device_id=(right,),
                device_id_type=pl.DeviceIdType.MESH,
            )
            rdma.start()
            rdma.wait()

        c_own = lax.rem(my + 1, N_DEV)
        acc = pacc_ref[pl.ds(c_own * CHUNK, CHUNK), :] + rs_recv_ref[N_DEV - 2]
        y = acc * jax.nn.sigmoid(acc)
        out_ref[pl.ds(c_own * CHUNK, CHUNK), :] = y
        send_ref[...] = y

        for s in range(N_DEV - 1):
            if s > 0:
                send_ref[...] = ag_recv_ref[s - 1]
            rdma = pltpu.make_async_remote_copy(
                src_ref=send_ref,
                dst_ref=ag_recv_ref.at[s],
                send_sem=ag_send_sems.at[s],
                recv_sem=ag_recv_sems.at[s],
                device_id=(right,),
                device_id_type=pl.DeviceIdType.MESH,
            )
            rdma.start()
            rdma.wait()
            c_in = lax.rem(my - s + N_DEV, N_DEV)
            out_ref[pl.ds(c_in * CHUNK, CHUNK), :] = ag_recv_ref[s]

    return pl.pallas_call(
        body,
        out_shape=jax.ShapeDtypeStruct((M, N), jnp.float32),
        in_specs=[
            pl.BlockSpec(memory_space=pltpu.VMEM),
            pl.BlockSpec(memory_space=pltpu.VMEM),
        ],
        out_specs=pl.BlockSpec(memory_space=pltpu.VMEM),
        scratch_shapes=[
            pltpu.VMEM((M, N), jnp.float32),
            pltpu.VMEM((CHUNK, N), jnp.float32),
            pltpu.VMEM((N_DEV - 1, CHUNK, N), jnp.float32),
            pltpu.VMEM((N_DEV - 1, CHUNK, N), jnp.float32),
            pltpu.SemaphoreType.DMA((N_DEV - 1,)),
            pltpu.SemaphoreType.DMA((N_DEV - 1,)),
            pltpu.SemaphoreType.DMA((N_DEV - 1,)),
            pltpu.SemaphoreType.DMA((N_DEV - 1,)),
        ],
    )(x, w_mat)


# device time: 60975 ns/iter; 1.9184x vs baseline; 1.9184x over previous
import jax
import jax.numpy as jnp
from jax import lax
from jax.experimental import pallas as pl
from jax.experimental.pallas import tpu as pltpu

N_DEV = 8
M = 1024
N = 1024
CHUNK = M // N_DEV
NSTREAM = 4
SIGMA = (1, 1, -1, -1)
CPS = N // NSTREAM
COL0 = (0, 256, 512, 768)
DEPTH = 3
T_RS = N_DEV - 1
T_TOT = 2 * (N_DEV - 1)


def kernel(x, w_mat):
    def body(
        x_ref,
        w_ref,
        out_ref,
        pacc_ref,
        sendbuf_ref,
        slots_ref,
        send_sems,
        recv_sems,
    ):
        my = lax.axis_index("i")

        def mod8(v):
            return lax.rem(v + 16, N_DEV)

        tgt = [mod8(my + SIGMA[k]) for k in range(NSTREAM)]

        pacc_ref[...] = jnp.dot(
            x_ref[...], w_ref[...], preferred_element_type=jnp.float32
        )

        def make(k, t, src):
            return pltpu.make_async_remote_copy(
                src_ref=src,
                dst_ref=slots_ref.at[k, t % DEPTH],
                send_sem=send_sems.at[k, t % DEPTH],
                recv_sem=recv_sems.at[k, t % DEPTH],
                device_id=(tgt[k],),
                device_id_type=pl.DeviceIdType.MESH,
            )

        def prev_src(k, t):
            if t >= 8:
                return slots_ref.at[k, (t - 1) % DEPTH]
            return sendbuf_ref.at[k]

        for t in range(T_TOT):
            for k in range(NSTREAM):
                sig = SIGMA[k]
                c0 = COL0[k]
                if t > 0:
                    make(k, t - 1, prev_src(k, t - 1)).wait()
                if t == 0:
                    sendbuf_ref[k] = pacc_ref[
                        pl.ds(my * CHUNK, CHUNK), c0 : c0 + CPS
                    ]
                    src = sendbuf_ref.at[k]
                elif t <= T_RS - 1:
                    c = mod8(my - sig * t)
                    sendbuf_ref[k] = (
                        pacc_ref[pl.ds(c * CHUNK, CHUNK), c0 : c0 + CPS]
                        + slots_ref[k, (t - 1) % DEPTH]
                    )
                    src = sendbuf_ref.at[k]
                elif t == T_RS:
                    c_own = mod8(my + sig)
                    acc = (
                        pacc_ref[pl.ds(c_own * CHUNK, CHUNK), c0 : c0 + CPS]
                        + slots_ref[k, (t - 1) % DEPTH]
                    )
                    y = acc * jax.nn.sigmoid(acc)
                    sendbuf_ref[k] = y
                    src = sendbuf_ref.at[k]
                else:
                    src = slots_ref.at[k, (t - 1) % DEPTH]
                make(k, t, src).start()
                if t == T_RS:
                    out_ref[
                        pl.ds(mod8(my + sig) * CHUNK, CHUNK), c0 : c0 + CPS
                    ] = sendbuf_ref[k]
                elif t > T_RS:
                    c_in = mod8(my - sig * (t - 1 - T_RS))
                    out_ref[pl.ds(c_in * CHUNK, CHUNK), c0 : c0 + CPS] = (
                        slots_ref[k, (t - 1) % DEPTH]
                    )

        for k in range(NSTREAM):
            sig = SIGMA[k]
            c0 = COL0[k]
            make(k, T_TOT - 1, prev_src(k, T_TOT - 1)).wait()
            c_in = mod8(my - sig * (N_DEV - 2))
            out_ref[pl.ds(c_in * CHUNK, CHUNK), c0 : c0 + CPS] = slots_ref[
                k, (T_TOT - 1) % DEPTH
            ]

    return pl.pallas_call(
        body,
        out_shape=jax.ShapeDtypeStruct((M, N), jnp.float32),
        in_specs=[
            pl.BlockSpec(memory_space=pltpu.VMEM),
            pl.BlockSpec(memory_space=pltpu.VMEM),
        ],
        out_specs=pl.BlockSpec(memory_space=pltpu.VMEM),
        scratch_shapes=[
            pltpu.VMEM((M, N), jnp.float32),
            pltpu.VMEM((NSTREAM, CHUNK, CPS), jnp.float32),
            pltpu.VMEM((NSTREAM, DEPTH, CHUNK, CPS), jnp.float32),
            pltpu.SemaphoreType.DMA((NSTREAM, DEPTH)),
            pltpu.SemaphoreType.DMA((NSTREAM, DEPTH)),
        ],
    )(x, w_mat)


# device time: 52683 ns/iter; 2.2203x vs baseline; 1.1574x over previous
import jax
import jax.numpy as jnp
from jax import lax
from jax.experimental import pallas as pl
from jax.experimental.pallas import tpu as pltpu

N_DEV = 8
M = 1024
N = 1024
CHUNK = M // N_DEV
NSTREAM = 8
SIGMA = (1, 1, 1, 1, -1, -1, -1, -1)
CPS = N // NSTREAM
COL0 = tuple(i * CPS for i in range(NSTREAM))
SLOT_DEPTH = 3
SEM_DEPTH = 2
T_RS = N_DEV - 1
T_TOT = 2 * (N_DEV - 1)


def kernel(x, w_mat):
    def body(
        x_ref,
        w_ref,
        out_ref,
        pacc_ref,
        sendbuf_ref,
        slots_ref,
        send_sems,
        recv_sems,
    ):
        my = lax.axis_index("i")

        def mod8(v):
            return lax.rem(v + 16, N_DEV)

        def perm(j):
            return jnp.where(j < 4, j, 11 - j)

        my_r = perm(my)
        tgt = [perm(mod8(my_r + SIGMA[k])) for k in range(NSTREAM)]

        def gemm_chunk(ring_j):
            r = perm(ring_j) * CHUNK
            pacc_ref[pl.ds(r, CHUNK), :] = jnp.dot(
                x_ref[pl.ds(r, CHUNK), :],
                w_ref[...],
                preferred_element_type=jnp.float32,
            )

        barrier = pltpu.get_barrier_semaphore()
        for nbr in (tgt[0], tgt[NSTREAM - 1]):
            pl.semaphore_signal(
                barrier,
                inc=1,
                device_id=(nbr,),
                device_id_type=pl.DeviceIdType.MESH,
            )
        gemm_chunk(my_r)
        pl.semaphore_wait(barrier, 2)

        def make(k, t):
            sig = SIGMA[k]
            c0 = COL0[k]
            if t < T_RS:
                src = sendbuf_ref.at[k]
                dst = slots_ref.at[k, t % SLOT_DEPTH]
            else:
                c = mod8(my_r + sig * (N_DEV - t))
                r = perm(c) * CHUNK
                src = out_ref.at[pl.ds(r, CHUNK), pl.ds(c0, CPS)]
                dst = out_ref.at[pl.ds(r, CHUNK), pl.ds(c0, CPS)]
            return pltpu.make_async_remote_copy(
                src_ref=src,
                dst_ref=dst,
                send_sem=send_sems.at[k, t % SEM_DEPTH],
                recv_sem=recv_sems.at[k, t % SEM_DEPTH],
                device_id=(tgt[k],),
                device_id_type=pl.DeviceIdType.MESH,
            )

        for t in range(T_TOT):
            for k in range(NSTREAM):
                sig = SIGMA[k]
                c0 = COL0[k]
                if t > 0:
                    make(k, t - 1).wait()
                if t == 0:
                    sendbuf_ref[k] = pacc_ref[
                        pl.ds(my * CHUNK, CHUNK), c0 : c0 + CPS
                    ]
                elif t < T_RS:
                    c = mod8(my_r - sig * t)
                    r = perm(c) * CHUNK
                    sendbuf_ref[k] = (
                        pacc_ref[pl.ds(r, CHUNK), c0 : c0 + CPS]
                        + slots_ref[k, (t - 1) % SLOT_DEPTH]
                    )
                elif t == T_RS:
                    c = mod8(my_r + sig)
                    r = perm(c) * CHUNK
                    acc = (
                        pacc_ref[pl.ds(r, CHUNK), c0 : c0 + CPS]
                        + slots_ref[k, (t - 1) % SLOT_DEPTH]
                    )
                    y = acc * jax.nn.sigmoid(acc)
                    out_ref[pl.ds(r, CHUNK), c0 : c0 + CPS] = y
                make(k, t).start()
            if t < 4:
                offs = [t + 1, -(t + 1)] if t < 3 else [4]
                for off in offs:
                    gemm_chunk(mod8(my_r + off))

        for k in range(NSTREAM):
            make(k, T_TOT - 1).wait()

    return pl.pallas_call(
        body,
        out_shape=jax.ShapeDtypeStruct((M, N), jnp.float32),
        in_specs=[
            pl.BlockSpec(memory_space=pltpu.VMEM),
            pl.BlockSpec(memory_space=pltpu.VMEM),
        ],
        out_specs=pl.BlockSpec(memory_space=pltpu.VMEM),
        scratch_shapes=[
            pltpu.VMEM((M, N), jnp.float32),
            pltpu.VMEM((NSTREAM, CHUNK, CPS), jnp.float32),
            pltpu.VMEM((NSTREAM, SLOT_DEPTH, CHUNK, CPS), jnp.float32),
            pltpu.SemaphoreType.DMA((NSTREAM, SEM_DEPTH)),
            pltpu.SemaphoreType.DMA((NSTREAM, SEM_DEPTH)),
        ],
        compiler_params=pltpu.CompilerParams(collective_id=0),
    )(x, w_mat)


# device time: 48963 ns/iter; 2.3890x vs baseline; 1.0760x over previous
import jax
import jax.numpy as jnp
from jax import lax
from jax.experimental import pallas as pl
from jax.experimental.pallas import tpu as pltpu

N_DEV = 8
M = 1024
N = 1024
CHUNK = M // N_DEV
NSTREAM = 8
BAND = CHUNK // NSTREAM
SIGMA = (1, -1, 1, -1, 1, -1, 1, -1)
BANDOF = (0, 4, 1, 5, 2, 6, 3, 7)
SLOT_DEPTH = 3
SEM_DEPTH = 2
T_RS = N_DEV - 1
T_TOT = 2 * (N_DEV - 1)


def kernel(x, w_mat):
    def body(
        x_ref,
        w_ref,
        out_ref,
        pacc_ref,
        sendbuf_ref,
        slots_ref,
        send_sems,
        recv_sems,
    ):
        my = lax.axis_index("i")

        def mod8(v):
            return lax.rem(v + 16, N_DEV)

        def perm(j):
            return jnp.where(j < 4, j, 11 - j)

        my_r = perm(my)
        tgt = [perm(mod8(my_r + SIGMA[k])) for k in range(NSTREAM)]

        def band_row(ring_j, k):
            return perm(ring_j) * CHUNK + BANDOF[k] * BAND

        def gemm_chunk(ring_j):
            r = perm(ring_j) * CHUNK
            pacc_ref[pl.ds(r, CHUNK), :] = jnp.dot(
                x_ref[pl.ds(r, CHUNK), :],
                w_ref[...],
                preferred_element_type=jnp.float32,
            )

        barrier = pltpu.get_barrier_semaphore()
        for nbr in (tgt[0], tgt[1]):
            pl.semaphore_signal(
                barrier,
                inc=1,
                device_id=(nbr,),
                device_id_type=pl.DeviceIdType.MESH,
            )
        gemm_chunk(my_r)
        pl.semaphore_wait(barrier, 2)

        def make(k, t):
            sig = SIGMA[k]
            if t < T_RS:
                src = sendbuf_ref.at[k]
                dst = slots_ref.at[k, t % SLOT_DEPTH]
            else:
                c = mod8(my_r + sig * (N_DEV - t))
                r = band_row(c, k)
                src = out_ref.at[pl.ds(r, BAND), :]
                dst = out_ref.at[pl.ds(r, BAND), :]
            return pltpu.make_async_remote_copy(
                src_ref=src,
                dst_ref=dst,
                send_sem=send_sems.at[k, t % SEM_DEPTH],
                recv_sem=recv_sems.at[k, t % SEM_DEPTH],
                device_id=(tgt[k],),
                device_id_type=pl.DeviceIdType.MESH,
            )

        for t in range(T_TOT):
            for k in range(NSTREAM):
                sig = SIGMA[k]
                if t > 0:
                    make(k, t - 1).wait()
                if t == 0:
                    sendbuf_ref[k] = pacc_ref[pl.ds(band_row(my_r, k), BAND), :]
                elif t < T_RS:
                    c = mod8(my_r - sig * t)
                    sendbuf_ref[k] = (
                        pacc_ref[pl.ds(band_row(c, k), BAND), :]
                        + slots_ref[k, (t - 1) % SLOT_DEPTH]
                    )
                elif t == T_RS:
                    c = mod8(my_r + sig)
                    r = band_row(c, k)
                    acc = (
                        pacc_ref[pl.ds(r, BAND), :]
                        + slots_ref[k, (t - 1) % SLOT_DEPTH]
                    )
                    y = acc * jax.nn.sigmoid(acc)
                    out_ref[pl.ds(r, BAND), :] = y
                make(k, t).start()
            if t < 4:
                offs = [t + 1, -(t + 1)] if t < 3 else [4]
                for off in offs:
                    gemm_chunk(mod8(my_r + off))

        for k in range(NSTREAM):
            make(k, T_TOT - 1).wait()

    return pl.pallas_call(
        body,
        out_shape=jax.ShapeDtypeStruct((M, N), jnp.float32),
        in_specs=[
            pl.BlockSpec(memory_space=pltpu.VMEM),
            pl.BlockSpec(memory_space=pltpu.VMEM),
        ],
        out_specs=pl.BlockSpec(memory_space=pltpu.VMEM),
        scratch_shapes=[
            pltpu.VMEM((M, N), jnp.float32),
            pltpu.VMEM((NSTREAM, BAND, N), jnp.float32),
            pltpu.VMEM((NSTREAM, SLOT_DEPTH, BAND, N), jnp.float32),
            pltpu.SemaphoreType.DMA((NSTREAM, SEM_DEPTH)),
            pltpu.SemaphoreType.DMA((NSTREAM, SEM_DEPTH)),
        ],
        compiler_params=pltpu.CompilerParams(collective_id=0),
    )(x, w_mat)
